# baseline (speedup 1.0000x reference)
import functools
import math

import jax
import jax.numpy as jnp
from jax import lax
from jax.experimental import pallas as pl
from jax.experimental.pallas import tpu as pltpu
from jax.experimental.pallas import tpu_sc as plsc

B = 16384
NC, NS = 2, 16
NW = NC * NS
BPW = B // NW
CHUNK = 128
NCH = BPW // CHUNK

D_ITEM, D_CAT = 32, 16
H, OUT = 256, 64
_BN = 1.0 / math.sqrt(1.0 + 1e-5)

_sc_mesh = plsc.VectorSubcoreMesh(
    core_axis_name="c", subcore_axis_name="s", num_cores=NC, num_subcores=NS)


def _sc_gather_body(c0, c1, c2, c3, t0, t1, t2, t3,
                    g0, g1, g2, g3,
                    i0, i1, i2, i3,
                    r0a, r0b, r3a, r3b, r1, r2,
                    s0a, s0b, s3a, s3b, s1, s2):
    wid = lax.axis_index("s") * NC + lax.axis_index("c")
    base = wid * BPW

    for cref, iref in zip((c0, c1, c2, c3), (i0, i1, i2, i3)):
        pltpu.sync_copy(cref.at[pl.ds(wid * NCH, NCH)], iref)

    item_bufs, item_sems = (r0a, r0b), (s0a, s0b)
    brand_bufs, brand_sems = (r3a, r3b), (s3a, s3b)
    hi = [None] * NCH
    hb = [None] * NCH
    for j in range(NCH):
        hi[j] = pltpu.async_copy(t0.at[i0.at[j]], item_bufs[j % 2],
                                 item_sems[j % 2])
        hb[j] = pltpu.async_copy(t3.at[i3.at[j]], brand_bufs[j % 2],
                                 brand_sems[j % 2])
        h1 = pltpu.async_copy(t1.at[i1.at[j]], r1, s1)
        h2 = pltpu.async_copy(t2.at[i2.at[j]], r2, s2)
        sl = pl.ds(base + j * CHUNK, CHUNK)
        h1.wait()
        pltpu.sync_copy(r1, g1.at[sl])
        h2.wait()
        pltpu.sync_copy(r2, g2.at[sl])
        if j > 0:
            slp = pl.ds(base + (j - 1) * CHUNK, CHUNK)
            hi[j - 1].wait()
            pltpu.sync_copy(item_bufs[(j - 1) % 2], g0.at[slp])
            hb[j - 1].wait()
            pltpu.sync_copy(brand_bufs[(j - 1) % 2], g3.at[slp])
    slp = pl.ds(base + (NCH - 1) * CHUNK, CHUNK)
    hi[NCH - 1].wait()
    pltpu.sync_copy(item_bufs[(NCH - 1) % 2], g0.at[slp])
    hb[NCH - 1].wait()
    pltpu.sync_copy(brand_bufs[(NCH - 1) % 2], g3.at[slp])


_sc_gather = pl.kernel(
    _sc_gather_body,
    out_type=[jax.ShapeDtypeStruct((B, 128), jnp.float32) for _ in range(4)],
    mesh=_sc_mesh,
    scratch_types=(
        [pltpu.VMEM((NCH, CHUNK), jnp.int32) for _ in range(4)]
        + [pltpu.VMEM((CHUNK, 128), jnp.float32) for _ in range(6)]
        + [pltpu.SemaphoreType.DMA for _ in range(6)]),
)


def _mlp_body(g0, g1, g2, g3, q0, q1, q2, q3, c4, dn, pr,
              w1a, w1b, w1c, w1d, w1e, w1f,
              b1, gm, bt, w2, b2, out):
    e0 = jnp.zeros_like(g0[:, 0:D_ITEM])
    for k in range(4):
        m = (q0[...] == k).astype(jnp.float32)
        e0 = e0 + m * g0[:, k * D_ITEM:(k + 1) * D_ITEM]
    e1 = jnp.zeros_like(g1[:, 0:D_CAT])
    e2 = jnp.zeros_like(g2[:, 0:D_CAT])
    e3 = jnp.zeros_like(g3[:, 0:D_CAT])
    for k in range(8):
        sl = slice(k * D_CAT, (k + 1) * D_CAT)
        e1 = e1 + (q1[...] == k).astype(jnp.float32) * g1[:, sl]
        e2 = e2 + (q2[...] == k).astype(jnp.float32) * g2[:, sl]
        e3 = e3 + (q3[...] == k).astype(jnp.float32) * g3[:, sl]
    rows = q0.shape[0]
    oh = (c4[...] == lax.broadcasted_iota(jnp.int32, (rows, 8), 1)
          ).astype(jnp.float32)
    e4 = jnp.dot(oh, pr[...], preferred_element_type=jnp.float32)

    h = jnp.dot(e0, w1a[...], preferred_element_type=jnp.float32)
    h = h + jnp.dot(e1, w1b[...], preferred_element_type=jnp.float32)
    h = h + jnp.dot(e2, w1c[...], preferred_element_type=jnp.float32)
    h = h + jnp.dot(e3, w1d[...], preferred_element_type=jnp.float32)
    h = h + jnp.dot(e4, w1e[...], preferred_element_type=jnp.float32)
    h = h + jnp.dot(dn[...], w1f[...], preferred_element_type=jnp.float32)
    h = (h + b1[...]) * (_BN * gm[...]) + bt[...]
    h = jnp.maximum(h, 0.0)
    o = jnp.dot(h, w2[...], preferred_element_type=jnp.float32) + b2[...]
    nrm = jnp.sqrt(jnp.sum(o * o, axis=1, keepdims=True))
    out[...] = o / jnp.maximum(nrm, 1e-12)


def _mlp(g0, g1, g2, g3, q0, q1, q2, q3, c4, dn, pr,
         w1a, w1b, w1c, w1d, w1e, w1f, b1, gm, bt, w2, b2, block_rows=2048):
    grid = (B // block_rows,)

    def row_spec(d):
        return pl.BlockSpec((block_rows, d), lambda i: (i, 0))

    def full_spec(shape):
        return pl.BlockSpec(shape, lambda i: (0, 0))

    return pl.pallas_call(
        _mlp_body,
        grid=grid,
        in_specs=[
            row_spec(128), row_spec(128), row_spec(128), row_spec(128),
            row_spec(1), row_spec(1), row_spec(1), row_spec(1),
            row_spec(1), row_spec(3),
            full_spec((8, 8)),
            full_spec((D_ITEM, H)), full_spec((D_CAT, H)),
            full_spec((D_CAT, H)), full_spec((D_CAT, H)),
            full_spec((8, H)), full_spec((3, H)),
            full_spec((1, H)), full_spec((1, H)), full_spec((1, H)),
            full_spec((H, OUT)), full_spec((1, OUT)),
        ],
        out_specs=pl.BlockSpec((block_rows, OUT), lambda i: (i, 0)),
        out_shape=jax.ShapeDtypeStruct((B, OUT), jnp.float32),
    )(g0, g1, g2, g3, q0, q1, q2, q3, c4, dn, pr,
      w1a, w1b, w1c, w1d, w1e, w1f, b1, gm, bt, w2, b2)


def kernel(item_cat, item_dense, item_emb, cat_l1_emb, cat_l2_emb,
           brand_emb, price_emb, W1, b1, gamma, beta, W2, b2):
    ic = item_cat.astype(jnp.int32)
    c0, c1, c2, c3, c4 = (ic[:, j] for j in range(5))

    item128 = item_emb.reshape(-1, 128)
    l1_128 = cat_l1_emb.reshape(-1, 128)
    l2_128 = cat_l2_emb.reshape(-1, 128)
    brand128 = brand_emb.reshape(-1, 128)
    shape2d = (B // CHUNK, CHUNK)

    g0, g1, g2, g3 = _sc_gather(
        (c0 >> 2).reshape(shape2d), (c1 >> 3).reshape(shape2d),
        (c2 >> 3).reshape(shape2d), (c3 >> 3).reshape(shape2d),
        item128, l1_128, l2_128, brand128)

    w1a = W1[0:32]
    w1b = W1[32:48]
    w1c = W1[48:64]
    w1d = W1[64:80]
    w1e = W1[80:88]
    w1f = W1[88:91]

    return _mlp(g0, g1, g2, g3,
                (c0 & 3).reshape(B, 1), (c1 & 7).reshape(B, 1),
                (c2 & 7).reshape(B, 1), (c3 & 7).reshape(B, 1),
                c4.reshape(B, 1), item_dense, price_emb,
                w1a, w1b, w1c, w1d, w1e, w1f,
                b1.reshape(1, H), gamma.reshape(1, H), beta.reshape(1, H),
                W2, b2.reshape(1, OUT))


# speedup vs baseline: 1.1184x; 1.1184x over previous
import functools
import math

import jax
import jax.numpy as jnp
from jax import lax
from jax.experimental import pallas as pl
from jax.experimental.pallas import tpu as pltpu
from jax.experimental.pallas import tpu_sc as plsc

B = 16384
NC, NS = 2, 16
NW = NC * NS
BPW = B // NW
CHUNK = 128
NCH = BPW // CHUNK
L = 16

D_ITEM, D_CAT = 32, 16
H, OUT = 256, 64
_BN = 1.0 / math.sqrt(1.0 + 1e-5)

_TAB = ((2, 3, D_ITEM), (3, 7, D_CAT), (3, 7, D_CAT), (3, 7, D_CAT))

_sc_mesh = plsc.VectorSubcoreMesh(
    core_axis_name="c", subcore_axis_name="s", num_cores=NC, num_subcores=NS)


def _sc_gather_body(c0, c1, c2, c3, c4, t0, t1, t2, t3, t4,
                    e0, e1, e2, e3, e4,
                    raw0, raw1, raw2, raw3, raw4,
                    si0, si1, si2, si3,
                    ba, bb, b4,
                    o32, o16, o16p,
                    sa, sb):
    wid = lax.axis_index("s") * NC + lax.axis_index("c")
    base = wid * BPW
    raws = (raw0, raw1, raw2, raw3, raw4)
    sidx = (si0, si1, si2, si3)
    bufs = (ba, bb)
    ehbm = (e0, e1, e2, e3)
    sems = (sa, sb)

    for cref, rref in zip((c0, c1, c2, c3, c4), raws):
        pltpu.sync_copy(cref.at[pl.ds(base, BPW)], rref)
    pltpu.sync_copy(t4, b4)

    for t in range(4):
        sh = _TAB[t][0]
        for j in range(NCH):
            for k in range(CHUNK // L):
                v = raws[t][pl.ds(j * CHUNK + k * L, L)]
                sidx[t][j, pl.ds(k * L, L)] = lax.shift_right_logical(
                    v, jnp.int32(sh))

    def extract(t, j, buf, out):
        _, msk, width = _TAB[t]

        def grp(g, carry):
            rows = lax.iota(jnp.int32, L) + g * L
            rv = raws[t][pl.ds(j * CHUNK + g * L, L)]
            colbase = lax.bitwise_and(rv, jnp.int32(msk)) * width
            for jj in range(width):
                x = plsc.load_gather(buf, [rows, colbase + jj])
                plsc.store_scatter(out, [rows, jnp.full((L,), jj, jnp.int32)],
                                   x)
            return carry

        lax.fori_loop(0, CHUNK // L, grp, 0)

    def extract_price(j, out):
        def grp(g, carry):
            rows = lax.iota(jnp.int32, L) + g * L
            rv = raws[4][pl.ds(j * CHUNK + g * L, L)]
            colbase = rv * D_CAT
            zero = jnp.zeros((L,), jnp.int32)
            for jj in range(D_CAT):
                x = plsc.load_gather(b4, [zero, colbase + jj])
                plsc.store_scatter(out, [rows, jnp.full((L,), jj, jnp.int32)],
                                   x)
            return carry

        lax.fori_loop(0, CHUNK // L, grp, 0)

    tabs = (t0, t1, t2, t3)
    steps = [(j, t) for j in range(NCH) for t in range(4)]
    h = [None, None]

    def fire(s):
        j, t = steps[s]
        h[s % 2] = pltpu.async_copy(tabs[t].at[sidx[t].at[j]], bufs[s % 2],
                                    sems[s % 2])

    def drain(s):
        j, t = steps[s]
        h[s % 2].wait()
        out = o32 if t == 0 else o16
        extract(t, j, bufs[s % 2], out)
        pltpu.sync_copy(out, ehbm[t].at[pl.ds(base + j * CHUNK, CHUNK)])

    fire(0)
    for j in range(NCH):
        extract_price(j, o16p)
        pltpu.sync_copy(o16p, e4.at[pl.ds(base + j * CHUNK, CHUNK)])
    for s in range(1, len(steps)):
        fire(s)
        drain(s - 1)
    drain(len(steps) - 1)


_sc_gather = pl.kernel(
    _sc_gather_body,
    out_type=[jax.ShapeDtypeStruct((B, D_ITEM), jnp.float32)]
    + [jax.ShapeDtypeStruct((B, D_CAT), jnp.float32) for _ in range(4)],
    mesh=_sc_mesh,
    scratch_types=(
        [pltpu.VMEM((BPW,), jnp.int32) for _ in range(5)]
        + [pltpu.VMEM((NCH, CHUNK), jnp.int32) for _ in range(4)]
        + [pltpu.VMEM((CHUNK, 128), jnp.float32) for _ in range(2)]
        + [pltpu.VMEM((1, 128), jnp.float32)]
        + [pltpu.VMEM((CHUNK, D_ITEM), jnp.float32)]
        + [pltpu.VMEM((CHUNK, D_CAT), jnp.float32) for _ in range(2)]
        + [pltpu.SemaphoreType.DMA for _ in range(2)]),
    compiler_params=pltpu.CompilerParams(needs_layout_passes=False),
)


def _mlp_body(e0, e1, e2, e3, e4, dnT, w1a, w1b, w1c, w1d, w1e, w1f,
              b1, gm, bt, w2t, b2, out):
    h = jnp.dot(e0[...], w1a[...], preferred_element_type=jnp.float32)
    h = h + jnp.dot(e1[...], w1b[...], preferred_element_type=jnp.float32)
    h = h + jnp.dot(e2[...], w1c[...], preferred_element_type=jnp.float32)
    h = h + jnp.dot(e3[...], w1d[...], preferred_element_type=jnp.float32)
    h = h + jnp.dot(e4[...], w1e[...], preferred_element_type=jnp.float32)
    h = h + lax.dot_general(dnT[...], w1f[...], (((0,), (0,)), ((), ())),
                            preferred_element_type=jnp.float32)
    h = (h + b1[...]) * (_BN * gm[...]) + bt[...]
    h = jnp.maximum(h, 0.0)
    o = lax.dot_general(h, w2t[...], (((1,), (1,)), ((), ())),
                        preferred_element_type=jnp.float32) + b2[...]
    nrm = jnp.sqrt(jnp.sum(o * o, axis=1, keepdims=True))
    out[...] = o / jnp.maximum(nrm, 1e-12)


def _mlp(e0, e1, e2, e3, e4, dnT, w1a, w1b, w1c, w1d, w1e, w1f,
         b1, gm, bt, w2t, b2, block_rows=2048):
    grid = (B // block_rows,)

    def row_spec(d):
        return pl.BlockSpec((block_rows, d), lambda i: (i, 0))

    def full_spec(shape):
        return pl.BlockSpec(shape, lambda i: (0,) * len(shape))

    return pl.pallas_call(
        _mlp_body,
        grid=grid,
        in_specs=[
            row_spec(D_ITEM), row_spec(D_CAT), row_spec(D_CAT),
            row_spec(D_CAT), row_spec(D_CAT),
            pl.BlockSpec((3, block_rows), lambda i: (0, i)),
            full_spec((D_ITEM, H)), full_spec((D_CAT, H)),
            full_spec((D_CAT, H)), full_spec((D_CAT, H)),
            full_spec((D_CAT, H)), full_spec((3, H)),
            full_spec((1, H)), full_spec((1, H)), full_spec((1, H)),
            full_spec((OUT, H)), full_spec((1, OUT)),
        ],
        out_specs=pl.BlockSpec((block_rows, OUT), lambda i: (i, 0)),
        out_shape=jax.ShapeDtypeStruct((B, OUT), jnp.float32),
    )(e0, e1, e2, e3, e4, dnT, w1a, w1b, w1c, w1d, w1e, w1f,
      b1, gm, bt, w2t, b2)


def kernel(item_cat, item_dense, item_emb, cat_l1_emb, cat_l2_emb,
           brand_emb, price_emb, W1, b1, gamma, beta, W2, b2):
    ic = item_cat.astype(jnp.int32)
    c0, c1, c2, c3, c4 = (ic[:, j] for j in range(5))

    item128 = item_emb.reshape(-1, 128)
    l1_128 = cat_l1_emb.reshape(-1, 128)
    l2_128 = cat_l2_emb.reshape(-1, 128)
    brand128 = brand_emb.reshape(-1, 128)
    price128 = jnp.pad(price_emb, ((0, 0), (0, 8))).reshape(1, 128)

    e0, e1, e2, e3, e4 = _sc_gather(
        c0, c1, c2, c3, c4, item128, l1_128, l2_128, brand128, price128)

    w1a = W1[0:32]
    w1b = W1[32:48]
    w1c = W1[48:64]
    w1d = W1[64:80]
    w1e = jnp.pad(W1[80:88], ((0, 8), (0, 0)))
    w1f = W1[88:91]

    return _mlp(e0, e1, e2, e3, e4, item_dense.T,
                w1a, w1b, w1c, w1d, w1e, w1f,
                b1.reshape(1, H), gamma.reshape(1, H), beta.reshape(1, H),
                W2.T, b2.reshape(1, OUT))
